# baseline (speedup 1.0000x reference)
import functools

import jax
import jax.numpy as jnp
from jax import lax
from jax.experimental import pallas as pl
from jax.experimental.pallas import tpu as pltpu
from jax.experimental.pallas import tpu_sc as plsc

DIM = 32
NROW = 16384
NCOL = 26
NUM_INDICES = NROW * NCOL
QROWS = 250000
NUM_CORES = 2
NUM_SUBCORES = 16
NW = NUM_CORES * NUM_SUBCORES
B_PER_W = NUM_INDICES // NW
BLK = 128
GCH = 256
BLKS_PER_CH = GCH // BLK
NCH = B_PER_W // GCH
BLOCKS_PER_W = B_PER_W // BLK
NBUF = 2

_mesh = plsc.VectorSubcoreMesh(core_axis_name="c", subcore_axis_name="s")


@functools.partial(
    pl.kernel,
    mesh=_mesh,
    out_type=jax.ShapeDtypeStruct((NCOL, DIM // 8, NROW // 128, 8, 128),
                                  jnp.float32),
    scratch_types=[
        pltpu.VMEM((GCH,), jnp.int32),
        pltpu.VMEM((GCH,), jnp.int32),
        pltpu.VMEM((GCH,), jnp.int32),
        pltpu.VMEM((GCH,), jnp.int32),
        pltpu.VMEM((GCH,), jnp.int32),
        pltpu.VMEM((GCH,), jnp.int32),
        pltpu.VMEM((GCH, 128), jnp.float32),
        pltpu.VMEM((GCH, 128), jnp.float32),
        pltpu.VMEM((DIM, BLK), jnp.float32),
        pltpu.VMEM((DIM, BLK), jnp.float32),
        pltpu.VMEM((DIM, BLK), jnp.float32),
        pltpu.VMEM((DIM, BLK), jnp.float32),
        pltpu.SemaphoreType.DMA,
        pltpu.SemaphoreType.DMA,
        pltpu.SemaphoreType.DMA,
        pltpu.SemaphoreType.DMA,
    ],
    compiler_params=pltpu.CompilerParams(needs_layout_passes=False),
)
def _gather_kernel(idx_hbm, table_hbm, out_hbm,
                   idx0, idx1, q0, q1, s0, s1, g0, g1,
                   c00, c01, c10, c11, gsem0, gsem1, wsem0, wsem1):
    idxs = (idx0, idx1)
    qs = (q0, q1)
    ss = (s0, s1)
    gath = (g0, g1)
    cbufs = ((c00, c01), (c10, c11))
    gsems = (gsem0, gsem1)
    wsems = (wsem0, wsem1)

    wid = lax.axis_index("s") * NUM_CORES + lax.axis_index("c")
    base = wid * B_PER_W
    iota = lax.iota(jnp.int32, 16)

    def stage(i, b):
        off = base + i * GCH
        pltpu.sync_copy(idx_hbm.at[pl.ds(off, GCH)], idxs[b])
        for g in range(GCH // 16):
            v = idxs[b][pl.ds(g * 16, 16)]
            qs[b][pl.ds(g * 16, 16)] = lax.shift_right_logical(v, 2)
            ss[b][pl.ds(g * 16, 16)] = lax.shift_left(
                lax.bitwise_and(v, 3), 5)
        pltpu.async_copy(table_hbm.at[qs[b]], gath[b], gsems[b])

    def visit(i, b):
        pltpu.make_async_copy(
            table_hbm.at[qs[b]], gath[b], gsems[b]).wait()

        @pl.when(i >= NBUF)
        def _():
            for blk in range(BLKS_PER_CH):
                for c1 in range(DIM // 8):
                    pltpu.make_async_copy(
                        cbufs[b][blk].at[pl.ds(c1 * 8, 8), :],
                        out_hbm.at[0, c1, 0], wsems[b]).wait()

        for blk in range(BLKS_PER_CH):
            block_id = wid * BLOCKS_PER_W + i * BLKS_PER_CH + blk
            j = block_id // 128
            b1 = block_id % 128
            cb = cbufs[b][blk]
            for g in range(BLK // 16):
                rows = blk * BLK + g * 16 + iota
                svec = ss[b][pl.ds(blk * BLK + g * 16, 16)]
                for c in range(DIM):
                    cb[c, pl.ds(g * 16, 16)] = plsc.load_gather(
                        gath[b], [rows, svec + c])
            for c1 in range(DIM // 8):
                pltpu.async_copy(
                    cb.at[pl.ds(c1 * 8, 8), :],
                    out_hbm.at[j, c1, b1], wsems[b])

        @pl.when(i + NBUF < NCH)
        def _():
            stage(i + NBUF, b)

    for b in range(NBUF):
        stage(b, b)

    def body(jj, carry):
        for b in range(NBUF):
            visit(jj * NBUF + b, b)
        return carry

    lax.fori_loop(0, NCH // NBUF, body, 0)

    for b in range(NBUF):
        for blk in range(BLKS_PER_CH):
            for c1 in range(DIM // 8):
                pltpu.make_async_copy(
                    cbufs[b][blk].at[pl.ds(c1 * 8, 8), :],
                    out_hbm.at[0, c1, 0], wsems[b]).wait()


def kernel(x, table):
    idx = jnp.transpose(x).reshape(-1)
    table_q = table.reshape(QROWS, 128)
    o5 = _gather_kernel(idx, table_q)
    return jnp.transpose(o5, (2, 4, 0, 1, 3)).reshape(NROW, NCOL, DIM)


# speedup vs baseline: 1.0913x; 1.0913x over previous
import functools

import jax
import jax.numpy as jnp
from jax import lax
from jax.experimental import pallas as pl
from jax.experimental.pallas import tpu as pltpu
from jax.experimental.pallas import tpu_sc as plsc

DIM = 32
NROW = 16384
NCOL = 26
NUM_INDICES = NROW * NCOL
NUM_CORES = 2
NUM_SUBCORES = 16
NW = NUM_CORES * NUM_SUBCORES
B_PER_W = NUM_INDICES // NW
BLK = 128
NCH = B_PER_W // BLK
NBUF = 4
LAG = 8

_mesh = plsc.VectorSubcoreMesh(core_axis_name="c", subcore_axis_name="s")


@functools.partial(
    pl.kernel,
    mesh=_mesh,
    out_type=jax.ShapeDtypeStruct((NCOL, DIM // 8, NROW // 128, 8, 128),
                                  jnp.float32),
    scratch_types=[
        pltpu.VMEM((NBUF, BLK), jnp.int32),
        pltpu.VMEM((NBUF, BLK, DIM), jnp.float32),
        pltpu.VMEM((NBUF, DIM, BLK), jnp.float32),
        pltpu.SemaphoreType.DMA,
        pltpu.SemaphoreType.DMA,
        pltpu.SemaphoreType.DMA,
        pltpu.SemaphoreType.DMA,
        pltpu.SemaphoreType.DMA,
        pltpu.SemaphoreType.DMA,
        pltpu.SemaphoreType.DMA,
        pltpu.SemaphoreType.DMA,
    ],
    compiler_params=pltpu.CompilerParams(
        use_tc_tiling_on_sc=False, needs_layout_passes=False),
)
def _gather_kernel(idx_hbm, table_hbm, out_hbm, idx_v, gath, cbuf,
                   g0, g1, g2, g3, w0, w1, w2, w3):
    gsems = (g0, g1, g2, g3)
    wsems = (w0, w1, w2, w3)

    wid = lax.axis_index("s") * NUM_CORES + lax.axis_index("c")
    base = wid * B_PER_W
    iota = lax.iota(jnp.int32, 16)

    def stage(i, b):
        pltpu.sync_copy(idx_hbm.at[pl.ds(base + i * BLK, BLK)], idx_v.at[b])
        pltpu.async_copy(table_hbm.at[idx_v.at[b]], gath.at[b], gsems[b])

    def visit(i, b):
        pltpu.make_async_copy(
            table_hbm.at[idx_v.at[b]], gath.at[b], gsems[b]).wait()

        @pl.when(i >= NBUF)
        def _():
            for c1 in range(DIM // 8):
                pltpu.make_async_copy(
                    cbuf.at[b, pl.ds(c1 * 8, 8), :],
                    out_hbm.at[0, c1, 0], wsems[b]).wait()

        gb = gath.at[b]
        cb = cbuf.at[b]
        for g in range(BLK // 16):
            rows = g * 16 + iota
            vals = {}
            for c in range(DIM + LAG):
                if c < DIM:
                    vals[c] = plsc.load_gather(
                        gb, [rows, jnp.full((16,), c, jnp.int32)])
                if c >= LAG:
                    cb[c - LAG, pl.ds(g * 16, 16)] = vals.pop(c - LAG)

        block_id = wid * NCH + i
        j = block_id // 128
        b1 = block_id % 128
        for c1 in range(DIM // 8):
            pltpu.async_copy(
                cb.at[pl.ds(c1 * 8, 8), :], out_hbm.at[j, c1, b1], wsems[b])

        @pl.when(i + NBUF < NCH)
        def _():
            stage(i + NBUF, b)

    for b in range(NBUF):
        stage(b, b)

    def body(jj, carry):
        for b in range(NBUF):
            visit(jj * NBUF + b, b)
        return carry

    lax.fori_loop(0, NCH // NBUF, body, 0)

    for b in range(NBUF):
        for c1 in range(DIM // 8):
            pltpu.make_async_copy(
                cbuf.at[b, pl.ds(c1 * 8, 8), :],
                out_hbm.at[0, c1, 0], wsems[b]).wait()


def kernel(x, table):
    idx = jnp.transpose(x).reshape(-1)
    o5 = _gather_kernel(idx, table)
    return jnp.transpose(o5, (2, 4, 0, 1, 3)).reshape(NROW, NCOL, DIM)
